# Initial kernel scaffold; baseline (speedup 1.0000x reference)
import jax
import jax.numpy as jnp
from jax.experimental import pallas as pl


def kernel(x, edge_index, W1, b1, W2, b2):
    raise NotImplementedError("write your pallas kernel here")


# speedup vs baseline: 12.1499x; 12.1499x over previous
import functools

import jax
import jax.numpy as jnp
from jax import lax
from jax.experimental import pallas as pl
from jax.experimental.pallas import tpu as pltpu
from jax.experimental.pallas import tpu_sc as plsc

N_NODES = 10000
D_FEAT = 128
HIDDEN = 128
N_CLASSES = 64
N_EDGES = 320000

NC = 2
NS = 16
NW = NC * NS
EPW = N_EDGES // NW
K = 80
CH = EPW // K
N_PAD = 10240
STRIPE = N_PAD // NS
BN = 2000


def _sc_mesh():
    return plsc.VectorSubcoreMesh(core_axis_name="c", subcore_axis_name="s")


def _deg_call(dst, ones_rows, zeros_stripe):
    @functools.partial(
        pl.kernel,
        mesh=_sc_mesh(),
        out_type=jax.ShapeDtypeStruct((NC, N_PAD, HIDDEN), jnp.float32),
        scratch_types=[
            pltpu.VMEM((K,), jnp.int32),
            pltpu.VMEM((K, HIDDEN), jnp.float32),
            pltpu.VMEM_SHARED((N_PAD, HIDDEN), jnp.float32),
        ],
    )
    def k(dst_hbm, ones_hbm, zeros_hbm, out_hbm, didx, ones_v, acc):
        cid = lax.axis_index("c")
        sid = lax.axis_index("s")
        base = (sid * NC + cid) * EPW
        stripe = pl.ds(sid * STRIPE, STRIPE)
        pltpu.sync_copy(ones_hbm, ones_v)
        pltpu.sync_copy(zeros_hbm, acc.at[stripe])
        plsc.subcore_barrier()

        def body(c, carry):
            pltpu.sync_copy(dst_hbm.at[pl.ds(base + c * K, K)], didx)
            pltpu.sync_copy(ones_v, acc.at[didx], add=True)
            return carry

        lax.fori_loop(0, CH, body, 0)
        plsc.subcore_barrier()
        pltpu.sync_copy(acc.at[stripe], out_hbm.at[cid, stripe])

    return k(dst, ones_rows, zeros_stripe)


def _agg_call(p, src, dst, zeros_stripe, d):
    @functools.partial(
        pl.kernel,
        mesh=_sc_mesh(),
        out_type=jax.ShapeDtypeStruct((NC, N_PAD, d), jnp.float32),
        scratch_types=[
            pltpu.VMEM((K,), jnp.int32),
            pltpu.VMEM((K,), jnp.int32),
            pltpu.VMEM((K, d), jnp.float32),
            pltpu.VMEM_SHARED((N_PAD, d), jnp.float32),
            pltpu.SemaphoreType.DMA,
        ],
    )
    def k(p_hbm, src_hbm, dst_hbm, zeros_hbm, out_hbm, sidx, didx, rows, acc, sem):
        cid = lax.axis_index("c")
        sid = lax.axis_index("s")
        base = (sid * NC + cid) * EPW
        stripe = pl.ds(sid * STRIPE, STRIPE)
        pltpu.sync_copy(zeros_hbm, acc.at[stripe])
        plsc.subcore_barrier()

        def body(c, carry):
            off = base + c * K
            pltpu.sync_copy(src_hbm.at[pl.ds(off, K)], sidx)
            pltpu.sync_copy(dst_hbm.at[pl.ds(off, K)], didx)
            pltpu.async_copy(p_hbm.at[sidx], rows, sem).wait()
            pltpu.sync_copy(rows, acc.at[didx], add=True)
            return carry

        lax.fori_loop(0, CH, body, 0)
        plsc.subcore_barrier()
        pltpu.sync_copy(acc.at[stripe], out_hbm.at[cid, stripe])

    return k(p, src, dst, zeros_stripe)


def _dinv_block(dp_ref):
    deg = dp_ref[0, :, 0:1] + dp_ref[1, :, 0:1] + 1.0
    return lax.rsqrt(deg)


def _tc_a_call(dp, x, w1):
    def body(dp_ref, x_ref, w_ref, p_ref):
        dinv = _dinv_block(dp_ref)
        p_ref[...] = jnp.dot(x_ref[...] * dinv, w_ref[...],
                             preferred_element_type=jnp.float32)

    return pl.pallas_call(
        body,
        grid=(N_NODES // BN,),
        in_specs=[
            pl.BlockSpec((NC, BN, HIDDEN), lambda i: (0, i, 0)),
            pl.BlockSpec((BN, D_FEAT), lambda i: (i, 0)),
            pl.BlockSpec((D_FEAT, HIDDEN), lambda i: (0, 0)),
        ],
        out_specs=pl.BlockSpec((BN, HIDDEN), lambda i: (i, 0)),
        out_shape=jax.ShapeDtypeStruct((N_NODES, HIDDEN), jnp.float32),
    )(dp, x, w1)


def _tc_b_call(a1, p1, dp, b1):
    def body(a_ref, p_ref, dp_ref, b_ref, o_ref):
        dinv = _dinv_block(dp_ref)
        s = (a_ref[0] + a_ref[1] + p_ref[...]) * dinv + b_ref[...]
        o_ref[...] = jnp.maximum(s, 0.0) * dinv

    return pl.pallas_call(
        body,
        grid=(N_NODES // BN,),
        in_specs=[
            pl.BlockSpec((NC, BN, HIDDEN), lambda i: (0, i, 0)),
            pl.BlockSpec((BN, HIDDEN), lambda i: (i, 0)),
            pl.BlockSpec((NC, BN, HIDDEN), lambda i: (0, i, 0)),
            pl.BlockSpec((1, HIDDEN), lambda i: (0, 0)),
        ],
        out_specs=pl.BlockSpec((BN, HIDDEN), lambda i: (i, 0)),
        out_shape=jax.ShapeDtypeStruct((N_NODES, HIDDEN), jnp.float32),
    )(a1, p1, dp, b1)


def _tc_c_call(a2, r2, dp, b2, w2):
    def body(a_ref, r_ref, dp_ref, b_ref, w_ref, lp_ref, lg_ref):
        dinv = _dinv_block(dp_ref)
        z = (a_ref[0] + a_ref[1] + r_ref[...]) * dinv
        logits = jnp.dot(z, w_ref[...],
                         preferred_element_type=jnp.float32) + b_ref[...]
        m = jnp.max(logits, axis=-1, keepdims=True)
        lse = m + jnp.log(jnp.sum(jnp.exp(logits - m), axis=-1, keepdims=True))
        lg_ref[...] = logits
        lp_ref[...] = logits - lse

    spec = pl.BlockSpec((BN, N_CLASSES), lambda i: (i, 0))
    return pl.pallas_call(
        body,
        grid=(N_NODES // BN,),
        in_specs=[
            pl.BlockSpec((NC, BN, HIDDEN), lambda i: (0, i, 0)),
            pl.BlockSpec((BN, HIDDEN), lambda i: (i, 0)),
            pl.BlockSpec((NC, BN, HIDDEN), lambda i: (0, i, 0)),
            pl.BlockSpec((1, N_CLASSES), lambda i: (0, 0)),
            pl.BlockSpec((HIDDEN, N_CLASSES), lambda i: (0, 0)),
        ],
        out_specs=[spec, spec],
        out_shape=[
            jax.ShapeDtypeStruct((N_NODES, N_CLASSES), jnp.float32),
            jax.ShapeDtypeStruct((N_NODES, N_CLASSES), jnp.float32),
        ],
    )(a2, r2, dp, b2, w2)


def kernel(x, edge_index, W1, b1, W2, b2):
    src = edge_index[0]
    dst = edge_index[1]
    z_h = jnp.zeros((STRIPE, HIDDEN), jnp.float32)

    ones_rows = jnp.ones((K, HIDDEN), jnp.float32)
    degc = _deg_call(dst, ones_rows, z_h)
    p1 = _tc_a_call(degc, x, W1)
    a1 = _agg_call(p1, src, dst, z_h, HIDDEN)
    r2 = _tc_b_call(a1, p1, degc, b1.reshape(1, HIDDEN))
    a2 = _agg_call(r2, src, dst, z_h, HIDDEN)
    log_probs, logits = _tc_c_call(a2, r2, degc, b2.reshape(1, N_CLASSES), W2)
    return (log_probs, logits)


# speedup vs baseline: 14.9388x; 1.2295x over previous
import functools

import jax
import jax.numpy as jnp
from jax import lax
from jax.experimental import pallas as pl
from jax.experimental.pallas import tpu as pltpu
from jax.experimental.pallas import tpu_sc as plsc

N_NODES = 10000
D_FEAT = 128
HIDDEN = 128
N_CLASSES = 64
N_EDGES = 320000

NC = 2
NS = 16
NW = NC * NS
EPW = N_EDGES // NW
K = 80
CH = EPW // K
N_PAD = 10240
STRIPE = N_PAD // NS
BN = 2000


def _sc_mesh():
    return plsc.VectorSubcoreMesh(core_axis_name="c", subcore_axis_name="s")


def _deg_call(dst, ones_rows, zeros_stripe):
    @functools.partial(
        pl.kernel,
        mesh=_sc_mesh(),
        out_type=jax.ShapeDtypeStruct((NC, N_PAD, HIDDEN), jnp.float32),
        scratch_types=[
            pltpu.VMEM((K,), jnp.int32),
            pltpu.VMEM((K,), jnp.int32),
            pltpu.VMEM((K, HIDDEN), jnp.float32),
            pltpu.VMEM_SHARED((N_PAD, HIDDEN), jnp.float32),
            pltpu.SemaphoreType.DMA,
        ],
    )
    def k(dst_hbm, ones_hbm, zeros_hbm, out_hbm, didx0, didx1, ones_v, acc, ssem):
        cid = lax.axis_index("c")
        sid = lax.axis_index("s")
        base = (sid * NC + cid) * EPW
        stripe = pl.ds(sid * STRIPE, STRIPE)
        didx = (didx0, didx1)
        pltpu.sync_copy(ones_hbm, ones_v)
        pltpu.sync_copy(zeros_hbm, acc.at[stripe])
        plsc.subcore_barrier()

        def icopy(cc, q):
            pltpu.sync_copy(dst_hbm.at[pl.ds(base + cc * K, K)], didx[q])

        def sstart(q):
            pltpu.async_copy(ones_v, acc.at[didx[q]], ssem, add=True)

        def swait(q):
            pltpu.make_async_copy(ones_v, acc.at[didx[q]], ssem).wait()

        icopy(0, 0)
        sstart(0)

        def body(t, carry):
            for q, off in ((1, 1), (0, 2)):
                cc = 2 * t + off
                icopy(cc, q)
                sstart(q)
                swait(1 - q)
            return carry

        lax.fori_loop(0, (CH - 1) // 2, body, 0)
        swait(0)
        plsc.subcore_barrier()
        pltpu.sync_copy(acc.at[stripe], out_hbm.at[cid, stripe])

    return k(dst, ones_rows, zeros_stripe)


def _agg_call(p, src, dst, zeros_stripe, d):
    @functools.partial(
        pl.kernel,
        mesh=_sc_mesh(),
        out_type=jax.ShapeDtypeStruct((NC, N_PAD, d), jnp.float32),
        scratch_types=[
            pltpu.VMEM((K,), jnp.int32),
            pltpu.VMEM((K,), jnp.int32),
            pltpu.VMEM((K, d), jnp.float32),
            pltpu.VMEM((K,), jnp.int32),
            pltpu.VMEM((K,), jnp.int32),
            pltpu.VMEM((K, d), jnp.float32),
            pltpu.VMEM_SHARED((N_PAD, d), jnp.float32),
            pltpu.SemaphoreType.DMA,
            pltpu.SemaphoreType.DMA,
        ],
    )
    def k(p_hbm, src_hbm, dst_hbm, zeros_hbm, out_hbm,
          sidx0, didx0, rows0, sidx1, didx1, rows1, acc, gsem, ssem):
        cid = lax.axis_index("c")
        sid = lax.axis_index("s")
        base = (sid * NC + cid) * EPW
        stripe = pl.ds(sid * STRIPE, STRIPE)
        sidx = (sidx0, sidx1)
        didx = (didx0, didx1)
        rows = (rows0, rows1)
        pltpu.sync_copy(zeros_hbm, acc.at[stripe])
        plsc.subcore_barrier()

        def icopy(cc, b):
            off = base + cc * K
            pltpu.sync_copy(src_hbm.at[pl.ds(off, K)], sidx[b])
            pltpu.sync_copy(dst_hbm.at[pl.ds(off, K)], didx[b])

        def gstart(b):
            pltpu.async_copy(p_hbm.at[sidx[b]], rows[b], gsem)

        def gwait(b):
            pltpu.make_async_copy(p_hbm.at[sidx[b]], rows[b], gsem).wait()

        def sstart(b):
            pltpu.async_copy(rows[b], acc.at[didx[b]], ssem, add=True)

        def swait(b):
            pltpu.make_async_copy(rows[b], acc.at[didx[b]], ssem).wait()

        icopy(0, 0)
        gstart(0)
        gwait(0)
        sstart(0)
        icopy(1, 1)
        gstart(1)

        def body(t, carry):
            for b, off in ((1, 1), (0, 2)):
                cc = 2 * t + off
                nb = 1 - b
                gwait(b)
                sstart(b)
                swait(nb)
                icopy(cc + 1, nb)
                gstart(nb)
            return carry

        lax.fori_loop(0, (CH - 3) // 2, body, 0)
        gwait(1)
        sstart(1)
        swait(0)
        icopy(CH - 1, 0)
        gstart(0)
        gwait(0)
        sstart(0)
        swait(1)
        swait(0)
        plsc.subcore_barrier()
        pltpu.sync_copy(acc.at[stripe], out_hbm.at[cid, stripe])

    return k(p, src, dst, zeros_stripe)


def _dinv_block(dp_ref):
    deg = dp_ref[0, :, 0:1] + dp_ref[1, :, 0:1] + 1.0
    return lax.rsqrt(deg)


def _tc_a_call(dp, x, w1):
    def body(dp_ref, x_ref, w_ref, p_ref):
        dinv = _dinv_block(dp_ref)
        p_ref[...] = jnp.dot(x_ref[...] * dinv, w_ref[...],
                             preferred_element_type=jnp.float32)

    return pl.pallas_call(
        body,
        grid=(N_NODES // BN,),
        in_specs=[
            pl.BlockSpec((NC, BN, HIDDEN), lambda i: (0, i, 0)),
            pl.BlockSpec((BN, D_FEAT), lambda i: (i, 0)),
            pl.BlockSpec((D_FEAT, HIDDEN), lambda i: (0, 0)),
        ],
        out_specs=pl.BlockSpec((BN, HIDDEN), lambda i: (i, 0)),
        out_shape=jax.ShapeDtypeStruct((N_NODES, HIDDEN), jnp.float32),
    )(dp, x, w1)


def _tc_b_call(a1, p1, dp, b1):
    def body(a_ref, p_ref, dp_ref, b_ref, o_ref):
        dinv = _dinv_block(dp_ref)
        s = (a_ref[0] + a_ref[1] + p_ref[...]) * dinv + b_ref[...]
        o_ref[...] = jnp.maximum(s, 0.0) * dinv

    return pl.pallas_call(
        body,
        grid=(N_NODES // BN,),
        in_specs=[
            pl.BlockSpec((NC, BN, HIDDEN), lambda i: (0, i, 0)),
            pl.BlockSpec((BN, HIDDEN), lambda i: (i, 0)),
            pl.BlockSpec((NC, BN, HIDDEN), lambda i: (0, i, 0)),
            pl.BlockSpec((1, HIDDEN), lambda i: (0, 0)),
        ],
        out_specs=pl.BlockSpec((BN, HIDDEN), lambda i: (i, 0)),
        out_shape=jax.ShapeDtypeStruct((N_NODES, HIDDEN), jnp.float32),
    )(a1, p1, dp, b1)


def _tc_c_call(a2, r2, dp, b2, w2):
    def body(a_ref, r_ref, dp_ref, b_ref, w_ref, lp_ref, lg_ref):
        dinv = _dinv_block(dp_ref)
        z = (a_ref[0] + a_ref[1] + r_ref[...]) * dinv
        logits = jnp.dot(z, w_ref[...],
                         preferred_element_type=jnp.float32) + b_ref[...]
        m = jnp.max(logits, axis=-1, keepdims=True)
        lse = m + jnp.log(jnp.sum(jnp.exp(logits - m), axis=-1, keepdims=True))
        lg_ref[...] = logits
        lp_ref[...] = logits - lse

    spec = pl.BlockSpec((BN, N_CLASSES), lambda i: (i, 0))
    return pl.pallas_call(
        body,
        grid=(N_NODES // BN,),
        in_specs=[
            pl.BlockSpec((NC, BN, HIDDEN), lambda i: (0, i, 0)),
            pl.BlockSpec((BN, HIDDEN), lambda i: (i, 0)),
            pl.BlockSpec((NC, BN, HIDDEN), lambda i: (0, i, 0)),
            pl.BlockSpec((1, N_CLASSES), lambda i: (0, 0)),
            pl.BlockSpec((HIDDEN, N_CLASSES), lambda i: (0, 0)),
        ],
        out_specs=[spec, spec],
        out_shape=[
            jax.ShapeDtypeStruct((N_NODES, N_CLASSES), jnp.float32),
            jax.ShapeDtypeStruct((N_NODES, N_CLASSES), jnp.float32),
        ],
    )(a2, r2, dp, b2, w2)


def kernel(x, edge_index, W1, b1, W2, b2):
    src = edge_index[0]
    dst = edge_index[1]
    z_h = jnp.zeros((STRIPE, HIDDEN), jnp.float32)

    ones_rows = jnp.ones((K, HIDDEN), jnp.float32)
    degc = _deg_call(dst, ones_rows, z_h)
    p1 = _tc_a_call(degc, x, W1)
    a1 = _agg_call(p1, src, dst, z_h, HIDDEN)
    r2 = _tc_b_call(a1, p1, degc, b1.reshape(1, HIDDEN))
    a2 = _agg_call(r2, src, dst, z_h, HIDDEN)
    log_probs, logits = _tc_c_call(a2, r2, degc, b2.reshape(1, N_CLASSES), W2)
    return (log_probs, logits)
